# Initial kernel scaffold; baseline (speedup 1.0000x reference)
import jax
import jax.numpy as jnp
from jax.experimental import pallas as pl


def kernel(x, edge_index, W1, b1, W2, b2):
    raise NotImplementedError("write your pallas kernel here")


# speedup vs baseline: 17.1590x; 17.1590x over previous
import functools

import jax
import jax.numpy as jnp
from jax import lax
from jax.experimental import pallas as pl
from jax.experimental.pallas import tpu as pltpu
from jax.experimental.pallas import tpu_sc as plsc

N = 10000
E = 320000
D = 128
H = 64
C = 7
K = 10
ALPHA = 0.1

FW = 16
NT = 16
NP = 10240
NPT = NP // NT
CH = 128
NCH = 157
EPT = CH * NCH
EPAD = NT * EPT
PADNODE = NP - 1

_mesh = plsc.VectorSubcoreMesh(core_axis_name="c", subcore_axis_name="s",
                               num_cores=1)
_sc_params = pltpu.CompilerParams(use_tc_tiling_on_sc=False)


@functools.partial(
    pl.kernel,
    mesh=_mesh,
    out_type=jax.ShapeDtypeStruct((NP, FW), jnp.float32),
    scratch_types=[
        pltpu.VMEM((NCH, CH), jnp.int32),
        pltpu.VMEM((CH, FW), jnp.float32),
        pltpu.VMEM((NPT, FW), jnp.float32),
        pltpu.VMEM_SHARED((NP, FW), jnp.float32),
    ],
    compiler_params=_sc_params,
)
def _deg_kernel(dst_hbm, deg_out, dst_idx, ones_t, zero_t, acc):
    w = lax.axis_index("s")
    sl = pl.ds(w * NPT, NPT)
    pltpu.sync_copy(dst_hbm.at[w], dst_idx)

    def fill(i, _):
        ones_t[i] = jnp.full((FW,), 1.0, jnp.float32)
        return 0
    lax.fori_loop(0, CH, fill, 0)

    def zfill(i, _):
        zero_t[i] = jnp.zeros((FW,), jnp.float32)
        return 0
    lax.fori_loop(0, NPT, zfill, 0)

    pltpu.sync_copy(zero_t, acc.at[sl])
    plsc.subcore_barrier()

    def chunk(c, _):
        pltpu.sync_copy(ones_t, acc.at[dst_idx.at[c]], add=True)
        return 0
    lax.fori_loop(0, NCH, chunk, 0)
    plsc.subcore_barrier()
    pltpu.sync_copy(acc.at[sl], deg_out.at[sl])


def _enc_body(x_ref, w1_ref, b1_ref, w2_ref, deg_ref, u0_ref, c1_ref):
    h = jnp.maximum(
        jnp.dot(x_ref[...], w1_ref[...], preferred_element_type=jnp.float32,
                precision=lax.Precision.HIGHEST)
        + b1_ref[...], 0.0)
    y = jnp.dot(h, w2_ref[...], preferred_element_type=jnp.float32,
                precision=lax.Precision.HIGHEST)
    deg = deg_ref[...] + 1.0
    i = pl.program_id(0)
    rows = lax.broadcasted_iota(jnp.int32, y.shape, 0) + i * y.shape[0]
    mask = rows < N
    dinv = lax.rsqrt(deg)
    u0_ref[...] = jnp.where(mask, dinv * y, 0.0)
    c1_ref[...] = jnp.where(mask, (1.0 - ALPHA) / deg, 0.0)


def _encoder(x_p, W1, b1r, W2p, deg16):
    blk = 512
    grid = NP // blk
    return pl.pallas_call(
        _enc_body,
        grid=(grid,),
        in_specs=[
            pl.BlockSpec((blk, D), lambda i: (i, 0)),
            pl.BlockSpec((D, H), lambda i: (0, 0)),
            pl.BlockSpec((1, H), lambda i: (0, 0)),
            pl.BlockSpec((H, FW), lambda i: (0, 0)),
            pl.BlockSpec((blk, FW), lambda i: (i, 0)),
        ],
        out_specs=[
            pl.BlockSpec((blk, FW), lambda i: (i, 0)),
            pl.BlockSpec((blk, FW), lambda i: (i, 0)),
        ],
        out_shape=[
            jax.ShapeDtypeStruct((NP, FW), jnp.float32),
            jax.ShapeDtypeStruct((NP, FW), jnp.float32),
        ],
    )(x_p, W1, b1r, W2p, deg16)


@functools.partial(
    pl.kernel,
    mesh=_mesh,
    out_type=jax.ShapeDtypeStruct((NP, FW), jnp.float32),
    scratch_types=[
        pltpu.VMEM((NCH, CH), jnp.int32),
        pltpu.VMEM((NCH, CH), jnp.int32),
        pltpu.VMEM((CH, FW), jnp.float32),
        pltpu.VMEM((NPT, FW), jnp.float32),
        pltpu.VMEM((NPT, FW), jnp.float32),
        pltpu.VMEM((NPT, FW), jnp.float32),
        pltpu.VMEM((NPT, FW), jnp.float32),
        pltpu.VMEM((NPT, FW), jnp.float32),
        pltpu.VMEM_SHARED((NP, FW), jnp.float32),
    ],
    compiler_params=_sc_params,
)
def _prop_kernel(src_hbm, dst_hbm, u0_hbm, c1_hbm, u_hbm,
                 src_idx, dst_idx, rows, u_t, c1_t, g_t, agg_t, zero_t, acc):
    w = lax.axis_index("s")
    sl = pl.ds(w * NPT, NPT)

    pltpu.sync_copy(src_hbm.at[w], src_idx)
    pltpu.sync_copy(dst_hbm.at[w], dst_idx)
    pltpu.sync_copy(u0_hbm.at[sl], u_t)
    pltpu.sync_copy(c1_hbm.at[sl], c1_t)

    def gfill(i, _):
        g_t[i] = u_t[i] * ALPHA
        zero_t[i] = jnp.zeros((FW,), jnp.float32)
        return 0
    lax.fori_loop(0, NPT, gfill, 0)

    pltpu.sync_copy(zero_t, acc.at[sl])
    pltpu.sync_copy(u_t, u_hbm.at[sl])
    plsc.subcore_barrier()

    def step(k, _):
        def chunk(c, _c):
            pltpu.sync_copy(u_hbm.at[src_idx.at[c]], rows)
            pltpu.sync_copy(rows, acc.at[dst_idx.at[c]], add=True)
            return 0
        lax.fori_loop(0, NCH, chunk, 0)
        plsc.subcore_barrier()

        pltpu.sync_copy(acc.at[sl], agg_t)
        pltpu.sync_copy(zero_t, acc.at[sl])

        def upd(i, _u):
            u_t[i] = c1_t[i] * (agg_t[i] + u_t[i]) + g_t[i]
            return 0
        lax.fori_loop(0, NPT, upd, 0)

        pltpu.sync_copy(u_t, u_hbm.at[sl])
        plsc.subcore_barrier()
        return 0

    lax.fori_loop(0, K, step, 0)


def _fin_body(u_ref, deg_ref, b2_ref, out_ref):
    out_ref[...] = jnp.sqrt(deg_ref[...] + 1.0) * u_ref[...] + b2_ref[...]


def _finish(u10, deg16, b2p):
    blk = 512
    return pl.pallas_call(
        _fin_body,
        grid=(NP // blk,),
        in_specs=[
            pl.BlockSpec((blk, FW), lambda i: (i, 0)),
            pl.BlockSpec((blk, FW), lambda i: (i, 0)),
            pl.BlockSpec((1, FW), lambda i: (0, 0)),
        ],
        out_specs=pl.BlockSpec((blk, FW), lambda i: (i, 0)),
        out_shape=jax.ShapeDtypeStruct((NP, FW), jnp.float32),
    )(u10, deg16, b2p)


def kernel(x, edge_index, W1, b1, W2, b2):
    src = edge_index[0]
    dst = edge_index[1]
    pad = jnp.full((EPAD - E,), PADNODE, dtype=jnp.int32)
    src3 = jnp.concatenate([src, pad]).reshape(NT, NCH, CH)
    dst3 = jnp.concatenate([dst, pad]).reshape(NT, NCH, CH)
    x_p = jnp.pad(x, ((0, NP - N), (0, 0)))
    b1r = b1.reshape(1, H)
    W2p = jnp.pad(W2, ((0, 0), (0, FW - C)))
    b2p = jnp.pad(b2, (0, FW - C)).reshape(1, FW)

    deg16 = _deg_kernel(dst3)
    u0, c1 = _encoder(x_p, W1, b1r, W2p, deg16)
    u10 = _prop_kernel(src3, dst3, u0, c1)
    out16 = _finish(u10, deg16, b2p)
    return out16[:N, :C]


# speedup vs baseline: 33.2085x; 1.9353x over previous
import functools

import jax
import jax.numpy as jnp
from jax import lax
from jax.experimental import pallas as pl
from jax.experimental.pallas import tpu as pltpu
from jax.experimental.pallas import tpu_sc as plsc

N = 10000
E = 320000
D = 128
H = 64
C = 7
K = 10
ALPHA = 0.1

FW = 16
NT = 16
NP = 10240
NPT = NP // NT
CH = 128
NCH = 160
NBUF = 8
EPT = CH * NCH
EPAD = NT * EPT
PADNODE = NP - 1

_mesh = plsc.VectorSubcoreMesh(core_axis_name="c", subcore_axis_name="s",
                               num_cores=1)
_sc_params = pltpu.CompilerParams(use_tc_tiling_on_sc=False)


@functools.partial(
    pl.kernel,
    mesh=_mesh,
    out_type=jax.ShapeDtypeStruct((NP, FW), jnp.float32),
    scratch_types=[
        pltpu.VMEM((NCH, CH), jnp.int32),
        pltpu.VMEM((CH, FW), jnp.float32),
        pltpu.VMEM((NPT, FW), jnp.float32),
        pltpu.VMEM_SHARED((NP, FW), jnp.float32),
    ],
    compiler_params=_sc_params,
)
def _deg_kernel(dst_hbm, deg_out, dst_idx, ones_t, zero_t, acc):
    w = lax.axis_index("s")
    sl = pl.ds(w * NPT, NPT)
    pltpu.sync_copy(dst_hbm.at[w], dst_idx)

    def fill(i, _):
        ones_t[i] = jnp.full((FW,), 1.0, jnp.float32)
        return 0
    lax.fori_loop(0, CH, fill, 0)

    def zfill(i, _):
        zero_t[i] = jnp.zeros((FW,), jnp.float32)
        return 0
    lax.fori_loop(0, NPT, zfill, 0)

    pltpu.sync_copy(zero_t, acc.at[sl])
    plsc.subcore_barrier()

    def chunk(c, _):
        pltpu.sync_copy(ones_t, acc.at[dst_idx.at[c]], add=True)
        return 0
    lax.fori_loop(0, NCH, chunk, 0)
    plsc.subcore_barrier()
    pltpu.sync_copy(acc.at[sl], deg_out.at[sl])


def _enc_body(x_ref, w1_ref, b1_ref, w2_ref, deg_ref, u0_ref, c1_ref):
    h = jnp.maximum(
        jnp.dot(x_ref[...], w1_ref[...], preferred_element_type=jnp.float32,
                precision=lax.Precision.HIGHEST)
        + b1_ref[...], 0.0)
    y = jnp.dot(h, w2_ref[...], preferred_element_type=jnp.float32,
                precision=lax.Precision.HIGHEST)
    deg = deg_ref[...] + 1.0
    i = pl.program_id(0)
    rows = lax.broadcasted_iota(jnp.int32, y.shape, 0) + i * y.shape[0]
    mask = rows < N
    dinv = lax.rsqrt(deg)
    u0_ref[...] = jnp.where(mask, dinv * y, 0.0)
    c1_ref[...] = jnp.where(mask, (1.0 - ALPHA) / deg, 0.0)


def _encoder(x_p, W1, b1r, W2p, deg16):
    blk = 512
    grid = NP // blk
    return pl.pallas_call(
        _enc_body,
        grid=(grid,),
        in_specs=[
            pl.BlockSpec((blk, D), lambda i: (i, 0)),
            pl.BlockSpec((D, H), lambda i: (0, 0)),
            pl.BlockSpec((1, H), lambda i: (0, 0)),
            pl.BlockSpec((H, FW), lambda i: (0, 0)),
            pl.BlockSpec((blk, FW), lambda i: (i, 0)),
        ],
        out_specs=[
            pl.BlockSpec((blk, FW), lambda i: (i, 0)),
            pl.BlockSpec((blk, FW), lambda i: (i, 0)),
        ],
        out_shape=[
            jax.ShapeDtypeStruct((NP, FW), jnp.float32),
            jax.ShapeDtypeStruct((NP, FW), jnp.float32),
        ],
    )(x_p, W1, b1r, W2p, deg16)


@functools.partial(
    pl.kernel,
    mesh=_mesh,
    out_type=jax.ShapeDtypeStruct((NP, FW), jnp.float32),
    scratch_types=[
        pltpu.VMEM((NCH, CH), jnp.int32),
        pltpu.VMEM((NCH, CH), jnp.int32),
        pltpu.VMEM((NBUF, CH, FW), jnp.float32),
        pltpu.VMEM((NPT, FW), jnp.float32),
        pltpu.VMEM((NPT, FW), jnp.float32),
        pltpu.VMEM((NPT, FW), jnp.float32),
        pltpu.VMEM((NPT, FW), jnp.float32),
        pltpu.VMEM((NPT, FW), jnp.float32),
        pltpu.VMEM_SHARED((NP, FW), jnp.float32),
        pltpu.SemaphoreType.DMA((NBUF,)),
    ],
    compiler_params=_sc_params,
)
def _prop_kernel(src_hbm, dst_hbm, u0_hbm, c1_hbm, u_hbm,
                 src_idx, dst_idx, rows, u_t, c1_t, g_t, agg_t, zero_t, acc,
                 gsem):
    w = lax.axis_index("s")
    sl = pl.ds(w * NPT, NPT)

    pltpu.sync_copy(src_hbm.at[w], src_idx)
    pltpu.sync_copy(dst_hbm.at[w], dst_idx)
    pltpu.sync_copy(u0_hbm.at[sl], u_t)
    pltpu.sync_copy(c1_hbm.at[sl], c1_t)

    def gfill(i, _):
        g_t[i] = u_t[i] * ALPHA
        zero_t[i] = jnp.zeros((FW,), jnp.float32)
        return 0
    lax.fori_loop(0, NPT, gfill, 0)

    pltpu.sync_copy(zero_t, acc.at[sl])
    pltpu.sync_copy(u_t, u_hbm.at[sl])
    plsc.subcore_barrier()

    def step(k, _):
        for b in range(NBUF):
            pltpu.async_copy(u_hbm.at[src_idx.at[b]], rows.at[b], gsem.at[b])

        def ring(j, _c):
            for b in range(NBUF):
                c = j * NBUF + b
                pltpu.make_async_copy(
                    u_hbm.at[src_idx.at[c]], rows.at[b], gsem.at[b]).wait()
                pltpu.sync_copy(rows.at[b], acc.at[dst_idx.at[c]], add=True)

                @pl.when(c + NBUF < NCH)
                def _():
                    pltpu.async_copy(u_hbm.at[src_idx.at[c + NBUF]],
                                     rows.at[b], gsem.at[b])
            return 0
        lax.fori_loop(0, NCH // NBUF, ring, 0)
        plsc.subcore_barrier()

        pltpu.sync_copy(acc.at[sl], agg_t)
        pltpu.sync_copy(zero_t, acc.at[sl])

        def upd(i, _u):
            u_t[i] = c1_t[i] * (agg_t[i] + u_t[i]) + g_t[i]
            return 0
        lax.fori_loop(0, NPT, upd, 0)

        pltpu.sync_copy(u_t, u_hbm.at[sl])
        plsc.subcore_barrier()
        return 0

    lax.fori_loop(0, K, step, 0)


def _fin_body(u_ref, deg_ref, b2_ref, out_ref):
    out_ref[...] = jnp.sqrt(deg_ref[...] + 1.0) * u_ref[...] + b2_ref[...]


def _finish(u10, deg16, b2p):
    blk = 512
    return pl.pallas_call(
        _fin_body,
        grid=(NP // blk,),
        in_specs=[
            pl.BlockSpec((blk, FW), lambda i: (i, 0)),
            pl.BlockSpec((blk, FW), lambda i: (i, 0)),
            pl.BlockSpec((1, FW), lambda i: (0, 0)),
        ],
        out_specs=pl.BlockSpec((blk, FW), lambda i: (i, 0)),
        out_shape=jax.ShapeDtypeStruct((NP, FW), jnp.float32),
    )(u10, deg16, b2p)


def kernel(x, edge_index, W1, b1, W2, b2):
    src = edge_index[0]
    dst = edge_index[1]
    pad = jnp.full((EPAD - E,), PADNODE, dtype=jnp.int32)
    src3 = jnp.concatenate([src, pad]).reshape(NT, NCH, CH)
    dst3 = jnp.concatenate([dst, pad]).reshape(NT, NCH, CH)
    x_p = jnp.pad(x, ((0, NP - N), (0, 0)))
    b1r = b1.reshape(1, H)
    W2p = jnp.pad(W2, ((0, 0), (0, FW - C)))
    b2p = jnp.pad(b2, (0, FW - C)).reshape(1, FW)

    deg16 = _deg_kernel(dst3)
    u0, c1 = _encoder(x_p, W1, b1r, W2p, deg16)
    u10 = _prop_kernel(src3, dst3, u0, c1)
    out16 = _finish(u10, deg16, b2p)
    return out16[:N, :C]


# speedup vs baseline: 33.2398x; 1.0009x over previous
import functools

import jax
import jax.numpy as jnp
from jax import lax
from jax.experimental import pallas as pl
from jax.experimental.pallas import tpu as pltpu
from jax.experimental.pallas import tpu_sc as plsc

N = 10000
E = 320000
D = 128
H = 64
C = 7
K = 10
ALPHA = 0.1

FW = 16
NT = 16
NP = 10240
NPT = NP // NT
CH = 128
NCH = 160
NBUF = 10
LAG = 5
EPT = CH * NCH
EPAD = NT * EPT
PADNODE = NP - 1

_mesh = plsc.VectorSubcoreMesh(core_axis_name="c", subcore_axis_name="s",
                               num_cores=1)
_sc_params = pltpu.CompilerParams(use_tc_tiling_on_sc=False)


@functools.partial(
    pl.kernel,
    mesh=_mesh,
    out_type=jax.ShapeDtypeStruct((NP, FW), jnp.float32),
    scratch_types=[
        pltpu.VMEM((NCH, CH), jnp.int32),
        pltpu.VMEM((CH, FW), jnp.float32),
        pltpu.VMEM((NPT, FW), jnp.float32),
        pltpu.VMEM_SHARED((NP, FW), jnp.float32),
    ],
    compiler_params=_sc_params,
)
def _deg_kernel(dst_hbm, deg_out, dst_idx, ones_t, zero_t, acc):
    w = lax.axis_index("s")
    sl = pl.ds(w * NPT, NPT)
    pltpu.sync_copy(dst_hbm.at[w], dst_idx)

    def fill(i, _):
        ones_t[i] = jnp.full((FW,), 1.0, jnp.float32)
        return 0
    lax.fori_loop(0, CH, fill, 0)

    def zfill(i, _):
        zero_t[i] = jnp.zeros((FW,), jnp.float32)
        return 0
    lax.fori_loop(0, NPT, zfill, 0)

    pltpu.sync_copy(zero_t, acc.at[sl])
    plsc.subcore_barrier()

    def chunk(c, _):
        pltpu.sync_copy(ones_t, acc.at[dst_idx.at[c]], add=True)
        return 0
    lax.fori_loop(0, NCH, chunk, 0)
    plsc.subcore_barrier()
    pltpu.sync_copy(acc.at[sl], deg_out.at[sl])


def _enc_body(x_ref, w1_ref, b1_ref, w2_ref, deg_ref, u0_ref, c1_ref):
    h = jnp.maximum(
        jnp.dot(x_ref[...], w1_ref[...], preferred_element_type=jnp.float32,
                precision=lax.Precision.HIGHEST)
        + b1_ref[...], 0.0)
    y = jnp.dot(h, w2_ref[...], preferred_element_type=jnp.float32,
                precision=lax.Precision.HIGHEST)
    deg = deg_ref[...] + 1.0
    i = pl.program_id(0)
    rows = lax.broadcasted_iota(jnp.int32, y.shape, 0) + i * y.shape[0]
    mask = rows < N
    dinv = lax.rsqrt(deg)
    u0_ref[...] = jnp.where(mask, dinv * y, 0.0)
    c1_ref[...] = jnp.where(mask, (1.0 - ALPHA) / deg, 0.0)


def _encoder(x_p, W1, b1r, W2p, deg16):
    blk = 512
    grid = NP // blk
    return pl.pallas_call(
        _enc_body,
        grid=(grid,),
        in_specs=[
            pl.BlockSpec((blk, D), lambda i: (i, 0)),
            pl.BlockSpec((D, H), lambda i: (0, 0)),
            pl.BlockSpec((1, H), lambda i: (0, 0)),
            pl.BlockSpec((H, FW), lambda i: (0, 0)),
            pl.BlockSpec((blk, FW), lambda i: (i, 0)),
        ],
        out_specs=[
            pl.BlockSpec((blk, FW), lambda i: (i, 0)),
            pl.BlockSpec((blk, FW), lambda i: (i, 0)),
        ],
        out_shape=[
            jax.ShapeDtypeStruct((NP, FW), jnp.float32),
            jax.ShapeDtypeStruct((NP, FW), jnp.float32),
        ],
    )(x_p, W1, b1r, W2p, deg16)


@functools.partial(
    pl.kernel,
    mesh=_mesh,
    out_type=jax.ShapeDtypeStruct((NP, FW), jnp.float32),
    scratch_types=[
        pltpu.VMEM((NCH, CH), jnp.int32),
        pltpu.VMEM((NCH, CH), jnp.int32),
        pltpu.VMEM((NBUF, CH, FW), jnp.float32),
        pltpu.VMEM((NPT, FW), jnp.float32),
        pltpu.VMEM((NPT, FW), jnp.float32),
        pltpu.VMEM((NPT, FW), jnp.float32),
        pltpu.VMEM((NPT, FW), jnp.float32),
        pltpu.VMEM((NPT, FW), jnp.float32),
        pltpu.VMEM_SHARED((NP, FW), jnp.float32),
        pltpu.SemaphoreType.DMA((NBUF,)),
        pltpu.SemaphoreType.DMA((NBUF,)),
    ],
    compiler_params=_sc_params,
)
def _prop_kernel(src_hbm, dst_hbm, u0_hbm, c1_hbm, u_hbm,
                 src_idx, dst_idx, rows, u_t, c1_t, g_t, agg_t, zero_t, acc,
                 gsem, ssem):
    w = lax.axis_index("s")
    sl = pl.ds(w * NPT, NPT)

    pltpu.sync_copy(src_hbm.at[w], src_idx)
    pltpu.sync_copy(dst_hbm.at[w], dst_idx)
    pltpu.sync_copy(u0_hbm.at[sl], u_t)
    pltpu.sync_copy(c1_hbm.at[sl], c1_t)

    def gfill(i, _):
        g_t[i] = u_t[i] * ALPHA
        zero_t[i] = jnp.zeros((FW,), jnp.float32)
        return 0
    lax.fori_loop(0, NPT, gfill, 0)

    pltpu.sync_copy(zero_t, acc.at[sl])
    pltpu.sync_copy(u_t, u_hbm.at[sl])
    plsc.subcore_barrier()

    def step(k, _):
        for b in range(LAG):
            pltpu.async_copy(u_hbm.at[src_idx.at[b]], rows.at[b], gsem.at[b])

        def ring(j, _c):
            for b in range(NBUF):
                c = j * NBUF + b
                bn = (b + LAG) % NBUF
                pltpu.make_async_copy(
                    u_hbm.at[src_idx.at[c]], rows.at[b], gsem.at[b]).wait()
                pltpu.async_copy(rows.at[b], acc.at[dst_idx.at[c]],
                                 ssem.at[b], add=True)

                @pl.when((c >= NBUF - LAG) & (c + LAG < NCH))
                def _():
                    pltpu.make_async_copy(
                        rows.at[bn], acc.at[dst_idx.at[0]], ssem.at[bn]).wait()

                @pl.when(c + LAG < NCH)
                def _():
                    pltpu.async_copy(u_hbm.at[src_idx.at[c + LAG]],
                                     rows.at[bn], gsem.at[bn])
            return 0
        lax.fori_loop(0, NCH // NBUF, ring, 0)
        for b in range(NBUF):
            pltpu.make_async_copy(
                rows.at[b], acc.at[dst_idx.at[0]], ssem.at[b]).wait()
        plsc.subcore_barrier()

        pltpu.sync_copy(acc.at[sl], agg_t)
        pltpu.sync_copy(zero_t, acc.at[sl])

        def upd(i, _u):
            for r in range(4):
                j = i * 4 + r
                u_t[j] = c1_t[j] * (agg_t[j] + u_t[j]) + g_t[j]
            return 0
        lax.fori_loop(0, NPT // 4, upd, 0)

        pltpu.sync_copy(u_t, u_hbm.at[sl])
        plsc.subcore_barrier()
        return 0

    lax.fori_loop(0, K, step, 0)


def _fin_body(u_ref, deg_ref, b2_ref, out_ref):
    out_ref[...] = jnp.sqrt(deg_ref[...] + 1.0) * u_ref[...] + b2_ref[...]


def _finish(u10, deg16, b2p):
    blk = 512
    return pl.pallas_call(
        _fin_body,
        grid=(NP // blk,),
        in_specs=[
            pl.BlockSpec((blk, FW), lambda i: (i, 0)),
            pl.BlockSpec((blk, FW), lambda i: (i, 0)),
            pl.BlockSpec((1, FW), lambda i: (0, 0)),
        ],
        out_specs=pl.BlockSpec((blk, FW), lambda i: (i, 0)),
        out_shape=jax.ShapeDtypeStruct((NP, FW), jnp.float32),
    )(u10, deg16, b2p)


def kernel(x, edge_index, W1, b1, W2, b2):
    src = edge_index[0]
    dst = edge_index[1]
    pad = jnp.full((EPAD - E,), PADNODE, dtype=jnp.int32)
    src3 = jnp.concatenate([src, pad]).reshape(NT, NCH, CH)
    dst3 = jnp.concatenate([dst, pad]).reshape(NT, NCH, CH)
    x_p = jnp.pad(x, ((0, NP - N), (0, 0)))
    b1r = b1.reshape(1, H)
    W2p = jnp.pad(W2, ((0, 0), (0, FW - C)))
    b2p = jnp.pad(b2, (0, FW - C)).reshape(1, FW)

    deg16 = _deg_kernel(dst3)
    u0, c1 = _encoder(x_p, W1, b1r, W2p, deg16)
    u10 = _prop_kernel(src3, dst3, u0, c1)
    out16 = _finish(u10, deg16, b2p)
    return out16[:N, :C]


# speedup vs baseline: 54.1860x; 1.6302x over previous
import functools

import jax
import jax.numpy as jnp
from jax import lax
from jax.experimental import pallas as pl
from jax.experimental.pallas import tpu as pltpu
from jax.experimental.pallas import tpu_sc as plsc

N = 10000
E = 320000
D = 128
H = 64
C = 7
K = 10
ALPHA = 0.1

FW = 16
NT = 16
NP = 10240
NPT = NP // NT
CH = 128
NCH = 160
NBUF = 10
LAG = 5
EPT = CH * NCH
EPAD = NT * EPT
PADNODE = NP - 1

_mesh = plsc.VectorSubcoreMesh(core_axis_name="c", subcore_axis_name="s",
                               num_cores=1)
_sc_params = pltpu.CompilerParams(use_tc_tiling_on_sc=False)


@functools.partial(
    pl.kernel,
    mesh=_mesh,
    out_type=jax.ShapeDtypeStruct((NP, FW), jnp.float32),
    scratch_types=[
        pltpu.VMEM((NCH, CH), jnp.int32),
        pltpu.VMEM((CH, FW), jnp.float32),
        pltpu.VMEM((NPT, FW), jnp.float32),
        pltpu.VMEM_SHARED((NP, FW), jnp.float32),
    ],
    compiler_params=_sc_params,
)
def _deg_kernel(dst_hbm, deg_out, dst_idx, ones_t, zero_t, acc):
    w = lax.axis_index("s")
    sl = pl.ds(w * NPT, NPT)
    pltpu.sync_copy(dst_hbm.at[w], dst_idx)

    def fill(i, _):
        ones_t[i] = jnp.full((FW,), 1.0, jnp.float32)
        return 0
    lax.fori_loop(0, CH, fill, 0)

    def zfill(i, _):
        zero_t[i] = jnp.zeros((FW,), jnp.float32)
        return 0
    lax.fori_loop(0, NPT, zfill, 0)

    pltpu.sync_copy(zero_t, acc.at[sl])
    plsc.subcore_barrier()

    def chunk(c, _):
        pltpu.sync_copy(ones_t, acc.at[dst_idx.at[c]], add=True)
        return 0
    lax.fori_loop(0, NCH, chunk, 0)
    plsc.subcore_barrier()
    pltpu.sync_copy(acc.at[sl], deg_out.at[sl])


def _enc_body(x_ref, w1_ref, b1_ref, w2_ref, deg_ref, u0_ref, c1_ref):
    h = jnp.maximum(
        jnp.dot(x_ref[...], w1_ref[...], preferred_element_type=jnp.float32,
                precision=lax.Precision.HIGHEST)
        + b1_ref[...], 0.0)
    y = jnp.dot(h, w2_ref[...], preferred_element_type=jnp.float32,
                precision=lax.Precision.HIGHEST)
    deg = deg_ref[...] + 1.0
    i = pl.program_id(0)
    rows = lax.broadcasted_iota(jnp.int32, y.shape, 0) + i * y.shape[0]
    mask = rows < N
    dinv = lax.rsqrt(deg)
    u0_ref[...] = jnp.where(mask, dinv * y, 0.0)
    c1_ref[...] = jnp.where(mask, (1.0 - ALPHA) / deg, 0.0)


def _encoder(x_p, W1, b1r, W2p, deg16):
    blk = 512
    grid = NP // blk
    return pl.pallas_call(
        _enc_body,
        grid=(grid,),
        in_specs=[
            pl.BlockSpec((blk, D), lambda i: (i, 0)),
            pl.BlockSpec((D, H), lambda i: (0, 0)),
            pl.BlockSpec((1, H), lambda i: (0, 0)),
            pl.BlockSpec((H, FW), lambda i: (0, 0)),
            pl.BlockSpec((blk, FW), lambda i: (i, 0)),
        ],
        out_specs=[
            pl.BlockSpec((blk, FW), lambda i: (i, 0)),
            pl.BlockSpec((blk, FW), lambda i: (i, 0)),
        ],
        out_shape=[
            jax.ShapeDtypeStruct((NP, FW), jnp.float32),
            jax.ShapeDtypeStruct((NP, FW), jnp.float32),
        ],
    )(x_p, W1, b1r, W2p, deg16)


@functools.partial(
    pl.kernel,
    mesh=_mesh,
    out_type=jax.ShapeDtypeStruct((NP, FW), jnp.float32),
    scratch_types=[
        pltpu.VMEM((NCH, CH), jnp.int32),
        pltpu.VMEM((NCH, CH), jnp.int32),
        pltpu.VMEM((NBUF, CH, FW), jnp.float32),
        pltpu.VMEM((NPT, FW), jnp.float32),
        pltpu.VMEM((NPT, FW), jnp.float32),
        pltpu.VMEM((NPT, FW), jnp.float32),
        pltpu.VMEM((NPT, FW), jnp.float32),
        pltpu.VMEM((NPT // 4, FW), jnp.float32),
        pltpu.VMEM_SHARED((NP, FW), jnp.float32),
        pltpu.VMEM_SHARED((NP, FW), jnp.float32),
        pltpu.SemaphoreType.DMA((NBUF,)),
        pltpu.SemaphoreType.DMA((NBUF,)),
    ],
    compiler_params=_sc_params,
)
def _prop_kernel(src_hbm, dst_hbm, u0_hbm, c1_hbm, u_hbm,
                 src_idx, dst_idx, rows, u_t, c1_t, g_t, agg_t, zero_t, acc,
                 u_sp, gsem, ssem):
    w = lax.axis_index("s")
    sl = pl.ds(w * NPT, NPT)

    pltpu.sync_copy(src_hbm.at[w], src_idx)
    pltpu.sync_copy(dst_hbm.at[w], dst_idx)
    pltpu.sync_copy(u0_hbm.at[sl], u_t)
    pltpu.sync_copy(c1_hbm.at[sl], c1_t)

    def gfill(i, _):
        g_t[i] = u_t[i] * ALPHA
        return 0
    lax.fori_loop(0, NPT, gfill, 0)

    def zfill(i, _):
        zero_t[i] = jnp.zeros((FW,), jnp.float32)
        return 0
    lax.fori_loop(0, NPT // 4, zfill, 0)

    def _zero_acc():
        for q in range(4):
            pltpu.sync_copy(
                zero_t, acc.at[pl.ds(w * NPT + q * (NPT // 4), NPT // 4)])

    _zero_acc()
    pltpu.sync_copy(u_t, u_sp.at[sl])
    plsc.subcore_barrier()

    def step(k, _):
        for b in range(LAG):
            pltpu.async_copy(u_sp.at[src_idx.at[b]], rows.at[b], gsem.at[b])

        def ring(j, _c):
            for b in range(NBUF):
                c = j * NBUF + b
                bn = (b + LAG) % NBUF
                pltpu.make_async_copy(
                    u_sp.at[src_idx.at[c]], rows.at[b], gsem.at[b]).wait()
                pltpu.async_copy(rows.at[b], acc.at[dst_idx.at[c]],
                                 ssem.at[b], add=True)

                @pl.when((c >= NBUF - LAG) & (c + LAG < NCH))
                def _():
                    pltpu.make_async_copy(
                        rows.at[bn], acc.at[dst_idx.at[0]], ssem.at[bn]).wait()

                @pl.when(c + LAG < NCH)
                def _():
                    pltpu.async_copy(u_sp.at[src_idx.at[c + LAG]],
                                     rows.at[bn], gsem.at[bn])
            return 0
        lax.fori_loop(0, NCH // NBUF, ring, 0)
        for b in range(NBUF):
            pltpu.make_async_copy(
                rows.at[b], acc.at[dst_idx.at[0]], ssem.at[b]).wait()
        plsc.subcore_barrier()

        pltpu.sync_copy(acc.at[sl], agg_t)
        _zero_acc()

        def upd(i, _u):
            for r in range(4):
                j = i * 4 + r
                u_t[j] = c1_t[j] * (agg_t[j] + u_t[j]) + g_t[j]
            return 0
        lax.fori_loop(0, NPT // 4, upd, 0)

        pltpu.sync_copy(u_t, u_sp.at[sl])

        @pl.when(k == K - 1)
        def _():
            pltpu.sync_copy(u_t, u_hbm.at[sl])
        plsc.subcore_barrier()
        return 0

    lax.fori_loop(0, K, step, 0)


def _fin_body(u_ref, deg_ref, b2_ref, out_ref):
    out_ref[...] = jnp.sqrt(deg_ref[...] + 1.0) * u_ref[...] + b2_ref[...]


def _finish(u10, deg16, b2p):
    blk = 512
    return pl.pallas_call(
        _fin_body,
        grid=(NP // blk,),
        in_specs=[
            pl.BlockSpec((blk, FW), lambda i: (i, 0)),
            pl.BlockSpec((blk, FW), lambda i: (i, 0)),
            pl.BlockSpec((1, FW), lambda i: (0, 0)),
        ],
        out_specs=pl.BlockSpec((blk, FW), lambda i: (i, 0)),
        out_shape=jax.ShapeDtypeStruct((NP, FW), jnp.float32),
    )(u10, deg16, b2p)


def kernel(x, edge_index, W1, b1, W2, b2):
    src = edge_index[0]
    dst = edge_index[1]
    pad = jnp.full((EPAD - E,), PADNODE, dtype=jnp.int32)
    src3 = jnp.concatenate([src, pad]).reshape(NT, NCH, CH)
    dst3 = jnp.concatenate([dst, pad]).reshape(NT, NCH, CH)
    x_p = jnp.pad(x, ((0, NP - N), (0, 0)))
    b1r = b1.reshape(1, H)
    W2p = jnp.pad(W2, ((0, 0), (0, FW - C)))
    b2p = jnp.pad(b2, (0, FW - C)).reshape(1, FW)

    deg16 = _deg_kernel(dst3)
    u0, c1 = _encoder(x_p, W1, b1r, W2p, deg16)
    u10 = _prop_kernel(src3, dst3, u0, c1)
    out16 = _finish(u10, deg16, b2p)
    return out16[:N, :C]


# speedup vs baseline: 67.0928x; 1.2382x over previous
import functools

import jax
import jax.numpy as jnp
from jax import lax
from jax.experimental import pallas as pl
from jax.experimental.pallas import tpu as pltpu
from jax.experimental.pallas import tpu_sc as plsc

N = 10000
E = 320000
D = 128
H = 64
C = 7
K = 10
ALPHA = 0.1

FW = 16
FWH = 8
NT = 16
NP = 10240
NPT = NP // NT
CH = 128
NCH = 160
NBUF = 10
LAG = 5
EPT = CH * NCH
EPAD = NT * EPT
PADNODE = NP - 1

_mesh = plsc.VectorSubcoreMesh(core_axis_name="c", subcore_axis_name="s",
                               num_cores=1)
_mesh2 = plsc.VectorSubcoreMesh(core_axis_name="c", subcore_axis_name="s",
                                num_cores=2)
_sc_params = pltpu.CompilerParams(use_tc_tiling_on_sc=False,
                                  needs_layout_passes=False)


@functools.partial(
    pl.kernel,
    mesh=_mesh,
    out_type=jax.ShapeDtypeStruct((NP, FW), jnp.float32),
    scratch_types=[
        pltpu.VMEM((NCH, CH), jnp.int32),
        pltpu.VMEM((CH, FW), jnp.float32),
        pltpu.VMEM((NPT, FW), jnp.float32),
        pltpu.VMEM_SHARED((NP, FW), jnp.float32),
    ],
    compiler_params=_sc_params,
)
def _deg_kernel(dst_hbm, deg_out, dst_idx, ones_t, zero_t, acc):
    w = lax.axis_index("s")
    sl = pl.ds(w * NPT, NPT)
    pltpu.sync_copy(dst_hbm.at[w], dst_idx)

    def fill(i, _):
        ones_t[i] = jnp.full((FW,), 1.0, jnp.float32)
        return 0
    lax.fori_loop(0, CH, fill, 0)

    def zfill(i, _):
        zero_t[i] = jnp.zeros((FW,), jnp.float32)
        return 0
    lax.fori_loop(0, NPT, zfill, 0)

    pltpu.sync_copy(zero_t, acc.at[sl])
    plsc.subcore_barrier()

    def chunk(c, _):
        pltpu.sync_copy(ones_t, acc.at[dst_idx.at[c]], add=True)
        return 0
    lax.fori_loop(0, NCH, chunk, 0)
    plsc.subcore_barrier()
    pltpu.sync_copy(acc.at[sl], deg_out.at[sl])


def _enc_body(x_ref, w1_ref, b1_ref, w2_ref, deg_ref, u0_ref, c1_ref, g_ref):
    h = jnp.maximum(
        jnp.dot(x_ref[...], w1_ref[...], preferred_element_type=jnp.float32,
                precision=lax.Precision.HIGHEST)
        + b1_ref[...], 0.0)
    y = jnp.dot(h, w2_ref[...], preferred_element_type=jnp.float32,
                precision=lax.Precision.HIGHEST)
    deg = deg_ref[...] + 1.0
    i = pl.program_id(0)
    rows = lax.broadcasted_iota(jnp.int32, y.shape, 0) + i * y.shape[0]
    mask = rows < N
    dinv = lax.rsqrt(deg)
    u0 = jnp.where(mask, dinv * y, 0.0)
    u0_ref[...] = u0
    c1_ref[...] = jnp.where(mask, (1.0 - ALPHA) / deg, 0.0)
    g_ref[...] = ALPHA * u0


def _encoder(x_p, W1, b1r, W2p, deg16):
    blk = 512
    grid = NP // blk
    return pl.pallas_call(
        _enc_body,
        grid=(grid,),
        in_specs=[
            pl.BlockSpec((blk, D), lambda i: (i, 0)),
            pl.BlockSpec((D, H), lambda i: (0, 0)),
            pl.BlockSpec((1, H), lambda i: (0, 0)),
            pl.BlockSpec((H, FW), lambda i: (0, 0)),
            pl.BlockSpec((blk, FW), lambda i: (i, 0)),
        ],
        out_specs=[
            pl.BlockSpec((blk, FW), lambda i: (i, 0)),
            pl.BlockSpec((blk, FW), lambda i: (i, 0)),
            pl.BlockSpec((blk, FW), lambda i: (i, 0)),
        ],
        out_shape=[
            jax.ShapeDtypeStruct((NP, FW), jnp.float32),
            jax.ShapeDtypeStruct((NP, FW), jnp.float32),
            jax.ShapeDtypeStruct((NP, FW), jnp.float32),
        ],
    )(x_p, W1, b1r, W2p, deg16)


@functools.partial(
    pl.kernel,
    mesh=_mesh2,
    out_type=[jax.ShapeDtypeStruct((NP, FWH), jnp.float32),
              jax.ShapeDtypeStruct((NP, FWH), jnp.float32)],
    scratch_types=[
        pltpu.VMEM((NCH, CH), jnp.int32),
        pltpu.VMEM((NCH, CH), jnp.int32),
        pltpu.VMEM((NBUF, CH, FWH), jnp.float32),
        pltpu.VMEM((NPT, FWH), jnp.float32),
        pltpu.VMEM((NPT, FWH), jnp.float32),
        pltpu.VMEM((NPT, FWH), jnp.float32),
        pltpu.VMEM((NPT, FWH), jnp.float32),
        pltpu.VMEM((NPT, FWH), jnp.float32),
        pltpu.VMEM_SHARED((NP, FWH), jnp.float32),
        pltpu.VMEM_SHARED((NP, FWH), jnp.float32),
        pltpu.SemaphoreType.DMA((NBUF,)),
        pltpu.SemaphoreType.DMA((NBUF,)),
    ],
    compiler_params=_sc_params,
)
def _prop_kernel(src_hbm, dst_hbm, u0a, u0b, ga, gb, c1h, zero8,
                 ua_out, ub_out,
                 src_idx, dst_idx, rows, u_t, c1_t, g_t, agg_t, zero_t, acc,
                 u_sp, gsem, ssem):
    cc = lax.axis_index("c")
    w = lax.axis_index("s")
    sl = pl.ds(w * NPT, NPT)

    pltpu.sync_copy(src_hbm.at[w], src_idx)
    pltpu.sync_copy(dst_hbm.at[w], dst_idx)

    @pl.when(cc == 0)
    def _():
        pltpu.sync_copy(u0a.at[sl], u_t)
        pltpu.sync_copy(ga.at[sl], g_t)

    @pl.when(cc == 1)
    def _():
        pltpu.sync_copy(u0b.at[sl], u_t)
        pltpu.sync_copy(gb.at[sl], g_t)

    pltpu.sync_copy(c1h.at[sl], c1_t)
    pltpu.sync_copy(zero8, zero_t)
    pltpu.sync_copy(zero_t, acc.at[sl])
    pltpu.sync_copy(u_t, u_sp.at[sl])
    plsc.subcore_barrier()

    lanes = lax.iota(jnp.int32, 16)

    def step(k, _):
        for b in range(LAG):
            pltpu.async_copy(u_sp.at[src_idx.at[b]], rows.at[b], gsem.at[b])

        def ring(j, _c):
            for b in range(NBUF):
                c = j * NBUF + b
                bn = (b + LAG) % NBUF
                pltpu.make_async_copy(
                    u_sp.at[src_idx.at[c]], rows.at[b], gsem.at[b]).wait()
                pltpu.async_copy(rows.at[b], acc.at[dst_idx.at[c]],
                                 ssem.at[b], add=True)

                @pl.when((c >= NBUF - LAG) & (c + LAG < NCH))
                def _():
                    pltpu.make_async_copy(
                        rows.at[bn], acc.at[dst_idx.at[0]], ssem.at[bn]).wait()

                @pl.when(c + LAG < NCH)
                def _():
                    pltpu.async_copy(u_sp.at[src_idx.at[c + LAG]],
                                     rows.at[bn], gsem.at[bn])
            return 0
        lax.fori_loop(0, NCH // NBUF, ring, 0)
        for b in range(NBUF):
            pltpu.make_async_copy(
                rows.at[b], acc.at[dst_idx.at[0]], ssem.at[b]).wait()
        plsc.subcore_barrier()

        pltpu.sync_copy(acc.at[sl], agg_t)
        pltpu.sync_copy(zero_t, acc.at[sl])

        def upd(i, _u):
            f = i * 16 + lanes
            r = lax.shift_right_logical(f, 3)
            col = lax.bitwise_and(f, 7)
            a = plsc.load_gather(agg_t, [r, col])
            u = plsc.load_gather(u_t, [r, col])
            c1v = plsc.load_gather(c1_t, [r, col])
            gv = plsc.load_gather(g_t, [r, col])
            plsc.store_scatter(u_t, [r, col], c1v * (a + u) + gv)
            return 0
        lax.fori_loop(0, NPT * FWH // 16, upd, 0)

        pltpu.sync_copy(u_t, u_sp.at[sl])
        plsc.subcore_barrier()
        return 0

    lax.fori_loop(0, K, step, 0)

    @pl.when(cc == 0)
    def _():
        pltpu.sync_copy(u_t, ua_out.at[sl])

    @pl.when(cc == 1)
    def _():
        pltpu.sync_copy(u_t, ub_out.at[sl])


def _fin_body(u_ref, deg_ref, b2_ref, out_ref):
    out_ref[...] = jnp.sqrt(deg_ref[...] + 1.0) * u_ref[...] + b2_ref[...]


def _finish(u10, deg16, b2p):
    blk = 512
    return pl.pallas_call(
        _fin_body,
        grid=(NP // blk,),
        in_specs=[
            pl.BlockSpec((blk, FW), lambda i: (i, 0)),
            pl.BlockSpec((blk, FW), lambda i: (i, 0)),
            pl.BlockSpec((1, FW), lambda i: (0, 0)),
        ],
        out_specs=pl.BlockSpec((blk, FW), lambda i: (i, 0)),
        out_shape=jax.ShapeDtypeStruct((NP, FW), jnp.float32),
    )(u10, deg16, b2p)


def kernel(x, edge_index, W1, b1, W2, b2):
    src = edge_index[0]
    dst = edge_index[1]
    pad = jnp.full((EPAD - E,), PADNODE, dtype=jnp.int32)
    src3 = jnp.concatenate([src, pad]).reshape(NT, NCH, CH)
    dst3 = jnp.concatenate([dst, pad]).reshape(NT, NCH, CH)
    x_p = jnp.pad(x, ((0, NP - N), (0, 0)))
    b1r = b1.reshape(1, H)
    W2p = jnp.pad(W2, ((0, 0), (0, FW - C)))
    b2p = jnp.pad(b2, (0, FW - C)).reshape(1, FW)

    deg16 = _deg_kernel(dst3)
    u0, c1, g16 = _encoder(x_p, W1, b1r, W2p, deg16)
    zero8 = jnp.zeros((NPT, FWH), jnp.float32)
    ua, ub = _prop_kernel(src3, dst3, u0[:, :FWH], u0[:, FWH:],
                          g16[:, :FWH], g16[:, FWH:], c1[:, :FWH], zero8)
    u10 = jnp.concatenate([ua, ub], axis=1)
    out16 = _finish(u10, deg16, b2p)
    return out16[:N, :C]


# speedup vs baseline: 67.8221x; 1.0109x over previous
import functools

import jax
import jax.numpy as jnp
from jax import lax
from jax.experimental import pallas as pl
from jax.experimental.pallas import tpu as pltpu
from jax.experimental.pallas import tpu_sc as plsc

N = 10000
E = 320000
D = 128
H = 64
C = 7
K = 10
ALPHA = 0.1

FW = 16
FWH = 8
NT = 16
NP = 10240
NPT = NP // NT
CH = 128
NCH = 160
NBUF = 10
LAG = 5
EPT = CH * NCH
EPAD = NT * EPT
PADNODE = NP - 1

_mesh = plsc.VectorSubcoreMesh(core_axis_name="c", subcore_axis_name="s",
                               num_cores=1)
_mesh2 = plsc.VectorSubcoreMesh(core_axis_name="c", subcore_axis_name="s",
                                num_cores=2)
_sc_params = pltpu.CompilerParams(use_tc_tiling_on_sc=False,
                                  needs_layout_passes=False)


@functools.partial(
    pl.kernel,
    mesh=_mesh,
    out_type=jax.ShapeDtypeStruct((NP, FWH), jnp.float32),
    scratch_types=[
        pltpu.VMEM((NCH, CH), jnp.int32),
        pltpu.VMEM((CH, FWH), jnp.float32),
        pltpu.VMEM((NPT, FWH), jnp.float32),
        pltpu.VMEM_SHARED((NP, FWH), jnp.float32),
    ],
    compiler_params=_sc_params,
)
def _deg_kernel(dst_hbm, ones8, zero8, deg_out, dst_idx, ones_t, zero_t, acc):
    w = lax.axis_index("s")
    sl = pl.ds(w * NPT, NPT)
    pltpu.sync_copy(dst_hbm.at[w], dst_idx)
    pltpu.sync_copy(ones8, ones_t)
    pltpu.sync_copy(zero8, zero_t)
    pltpu.sync_copy(zero_t, acc.at[sl])
    plsc.subcore_barrier()

    def chunk(c, _):
        pltpu.sync_copy(ones_t, acc.at[dst_idx.at[c]], add=True)
        return 0
    lax.fori_loop(0, NCH, chunk, 0)
    plsc.subcore_barrier()
    pltpu.sync_copy(acc.at[sl], deg_out.at[sl])


def _enc_body(x_ref, w1_ref, b1_ref, w2_ref, deg_ref, u0_ref, c1_ref, g_ref):
    h = jnp.maximum(
        jnp.dot(x_ref[...], w1_ref[...], preferred_element_type=jnp.float32,
                precision=lax.Precision.HIGHEST)
        + b1_ref[...], 0.0)
    y = jnp.dot(h, w2_ref[...], preferred_element_type=jnp.float32,
                precision=lax.Precision.HIGHEST)
    d8 = deg_ref[...]
    deg = jnp.concatenate([d8, d8], axis=1) + 1.0
    i = pl.program_id(0)
    rows = lax.broadcasted_iota(jnp.int32, y.shape, 0) + i * y.shape[0]
    mask = rows < N
    dinv = lax.rsqrt(deg)
    u0 = jnp.where(mask, dinv * y, 0.0)
    u0_ref[...] = u0
    c1_ref[...] = jnp.where(mask, (1.0 - ALPHA) / deg, 0.0)
    g_ref[...] = ALPHA * u0


def _encoder(x_p, W1, b1r, W2p, deg16):
    blk = 512
    grid = NP // blk
    return pl.pallas_call(
        _enc_body,
        grid=(grid,),
        in_specs=[
            pl.BlockSpec((blk, D), lambda i: (i, 0)),
            pl.BlockSpec((D, H), lambda i: (0, 0)),
            pl.BlockSpec((1, H), lambda i: (0, 0)),
            pl.BlockSpec((H, FW), lambda i: (0, 0)),
            pl.BlockSpec((blk, FWH), lambda i: (i, 0)),
        ],
        out_specs=[
            pl.BlockSpec((blk, FW), lambda i: (i, 0)),
            pl.BlockSpec((blk, FW), lambda i: (i, 0)),
            pl.BlockSpec((blk, FW), lambda i: (i, 0)),
        ],
        out_shape=[
            jax.ShapeDtypeStruct((NP, FW), jnp.float32),
            jax.ShapeDtypeStruct((NP, FW), jnp.float32),
            jax.ShapeDtypeStruct((NP, FW), jnp.float32),
        ],
    )(x_p, W1, b1r, W2p, deg16)


@functools.partial(
    pl.kernel,
    mesh=_mesh2,
    out_type=[jax.ShapeDtypeStruct((NP, FWH), jnp.float32),
              jax.ShapeDtypeStruct((NP, FWH), jnp.float32)],
    scratch_types=[
        pltpu.VMEM((NCH, CH), jnp.int32),
        pltpu.VMEM((NCH, CH), jnp.int32),
        pltpu.VMEM((NBUF, CH, FWH), jnp.float32),
        pltpu.VMEM((NPT, FWH), jnp.float32),
        pltpu.VMEM((NPT, FWH), jnp.float32),
        pltpu.VMEM((NPT, FWH), jnp.float32),
        pltpu.VMEM((NPT, FWH), jnp.float32),
        pltpu.VMEM((NPT, FWH), jnp.float32),
        pltpu.VMEM_SHARED((NP, FWH), jnp.float32),
        pltpu.VMEM_SHARED((NP, FWH), jnp.float32),
        pltpu.SemaphoreType.DMA((NBUF,)),
        pltpu.SemaphoreType.DMA((NBUF,)),
    ],
    compiler_params=_sc_params,
)
def _prop_kernel(src_hbm, dst_hbm, u0a, u0b, ga, gb, c1h, zero8,
                 ua_out, ub_out,
                 src_idx, dst_idx, rows, u_t, c1_t, g_t, agg_t, zero_t, acc,
                 u_sp, gsem, ssem):
    cc = lax.axis_index("c")
    w = lax.axis_index("s")
    sl = pl.ds(w * NPT, NPT)

    pltpu.sync_copy(src_hbm.at[w], src_idx)
    pltpu.sync_copy(dst_hbm.at[w], dst_idx)

    @pl.when(cc == 0)
    def _():
        pltpu.sync_copy(u0a.at[sl], u_t)
        pltpu.sync_copy(ga.at[sl], g_t)

    @pl.when(cc == 1)
    def _():
        pltpu.sync_copy(u0b.at[sl], u_t)
        pltpu.sync_copy(gb.at[sl], g_t)

    pltpu.sync_copy(c1h.at[sl], c1_t)
    pltpu.sync_copy(zero8, zero_t)
    pltpu.sync_copy(zero_t, acc.at[sl])
    pltpu.sync_copy(u_t, u_sp.at[sl])
    plsc.subcore_barrier()

    lanes = lax.iota(jnp.int32, 16)

    def step(k, _):
        for b in range(LAG):
            pltpu.async_copy(u_sp.at[src_idx.at[b]], rows.at[b], gsem.at[b])

        def ring(j, _c):
            for b in range(NBUF):
                c = j * NBUF + b
                bn = (b + LAG) % NBUF
                pltpu.make_async_copy(
                    u_sp.at[src_idx.at[c]], rows.at[b], gsem.at[b]).wait()
                pltpu.async_copy(rows.at[b], acc.at[dst_idx.at[c]],
                                 ssem.at[b], add=True)

                @pl.when((c >= NBUF - LAG) & (c + LAG < NCH))
                def _():
                    pltpu.make_async_copy(
                        rows.at[bn], acc.at[dst_idx.at[0]], ssem.at[bn]).wait()

                @pl.when(c + LAG < NCH)
                def _():
                    pltpu.async_copy(u_sp.at[src_idx.at[c + LAG]],
                                     rows.at[bn], gsem.at[bn])
            return 0
        lax.fori_loop(0, NCH // NBUF, ring, 0)
        for b in range(NBUF):
            pltpu.make_async_copy(
                rows.at[b], acc.at[dst_idx.at[0]], ssem.at[b]).wait()
        plsc.subcore_barrier()

        pltpu.sync_copy(acc.at[sl], agg_t)
        pltpu.sync_copy(zero_t, acc.at[sl])

        def upd(i, _u):
            f = i * 16 + lanes
            r = lax.shift_right_logical(f, 3)
            col = lax.bitwise_and(f, 7)
            a = plsc.load_gather(agg_t, [r, col])
            u = plsc.load_gather(u_t, [r, col])
            c1v = plsc.load_gather(c1_t, [r, col])
            gv = plsc.load_gather(g_t, [r, col])
            plsc.store_scatter(u_t, [r, col], c1v * (a + u) + gv)
            return 0
        lax.fori_loop(0, NPT * FWH // 16, upd, 0)

        pltpu.sync_copy(u_t, u_sp.at[sl])
        plsc.subcore_barrier()
        return 0

    lax.fori_loop(0, K, step, 0)

    @pl.when(cc == 0)
    def _():
        pltpu.sync_copy(u_t, ua_out.at[sl])

    @pl.when(cc == 1)
    def _():
        pltpu.sync_copy(u_t, ub_out.at[sl])


def _fin_body(u_ref, deg_ref, b2_ref, out_ref):
    d8 = deg_ref[...]
    deg = jnp.concatenate([d8, d8], axis=1) + 1.0
    out_ref[...] = jnp.sqrt(deg) * u_ref[...] + b2_ref[...]


def _finish(u10, deg16, b2p):
    blk = 512
    return pl.pallas_call(
        _fin_body,
        grid=(NP // blk,),
        in_specs=[
            pl.BlockSpec((blk, FW), lambda i: (i, 0)),
            pl.BlockSpec((blk, FWH), lambda i: (i, 0)),
            pl.BlockSpec((1, FW), lambda i: (0, 0)),
        ],
        out_specs=pl.BlockSpec((blk, FW), lambda i: (i, 0)),
        out_shape=jax.ShapeDtypeStruct((NP, FW), jnp.float32),
    )(u10, deg16, b2p)


def kernel(x, edge_index, W1, b1, W2, b2):
    src = edge_index[0]
    dst = edge_index[1]
    pad = jnp.full((EPAD - E,), PADNODE, dtype=jnp.int32)
    src3 = jnp.concatenate([src, pad]).reshape(NT, NCH, CH)
    dst3 = jnp.concatenate([dst, pad]).reshape(NT, NCH, CH)
    x_p = jnp.pad(x, ((0, NP - N), (0, 0)))
    b1r = b1.reshape(1, H)
    W2p = jnp.pad(W2, ((0, 0), (0, FW - C)))
    b2p = jnp.pad(b2, (0, FW - C)).reshape(1, FW)

    zero8 = jnp.zeros((NPT, FWH), jnp.float32)
    ones8 = jnp.ones((CH, FWH), jnp.float32)
    deg8 = _deg_kernel(dst3, ones8, zero8)
    u0, c1, g16 = _encoder(x_p, W1, b1r, W2p, deg8)
    ua, ub = _prop_kernel(src3, dst3, u0[:, :FWH], u0[:, FWH:],
                          g16[:, :FWH], g16[:, FWH:], c1[:, :FWH], zero8)
    u10 = jnp.concatenate([ua, ub], axis=1)
    out16 = _finish(u10, deg8, b2p)
    return out16[:N, :C]


# speedup vs baseline: 79.2410x; 1.1684x over previous
import functools

import jax
import jax.numpy as jnp
from jax import lax
from jax.experimental import pallas as pl
from jax.experimental.pallas import tpu as pltpu
from jax.experimental.pallas import tpu_sc as plsc

N = 10000
E = 320000
D = 128
H = 64
C = 7
K = 10
ALPHA = 0.1

FW = 16
FWH = 8
NT = 16
NP = 10240
NPT = NP // NT
CH = 128
NCH = 160
NBUF = 10
LAG = 5
EPT = CH * NCH
EPAD = NT * EPT
PADNODE = NP - 1
NV = NPT * FWH // 16

_mesh = plsc.VectorSubcoreMesh(core_axis_name="c", subcore_axis_name="s",
                               num_cores=1)
_sc_params = pltpu.CompilerParams(use_tc_tiling_on_sc=False,
                                  needs_layout_passes=False)


def _enc_body(x_ref, w1_ref, b1_ref, w2_ref, y_ref):
    h = jnp.maximum(
        jnp.dot(x_ref[...], w1_ref[...], preferred_element_type=jnp.float32,
                precision=lax.Precision.HIGHEST)
        + b1_ref[...], 0.0)
    y = jnp.dot(h, w2_ref[...], preferred_element_type=jnp.float32,
                precision=lax.Precision.HIGHEST)
    i = pl.program_id(0)
    rows = lax.broadcasted_iota(jnp.int32, y.shape, 0) + i * y.shape[0]
    y_ref[...] = jnp.where(rows < N, y, 0.0)


def _encoder(x_p, W1, b1r, W2p):
    blk = 512
    return pl.pallas_call(
        _enc_body,
        grid=(NP // blk,),
        in_specs=[
            pl.BlockSpec((blk, D), lambda i: (i, 0)),
            pl.BlockSpec((D, H), lambda i: (0, 0)),
            pl.BlockSpec((1, H), lambda i: (0, 0)),
            pl.BlockSpec((H, FW), lambda i: (0, 0)),
        ],
        out_specs=pl.BlockSpec((blk, FW), lambda i: (i, 0)),
        out_shape=jax.ShapeDtypeStruct((NP, FW), jnp.float32),
    )(x_p, W1, b1r, W2p)


def _rsqrt16(d):
    i = plsc.bitcast(d, jnp.int32)
    i = jnp.int32(0x5F3759DF) - lax.shift_right_logical(i, 1)
    y = plsc.bitcast(i, jnp.float32)
    for _ in range(3):
        y = y * (1.5 - 0.5 * d * y * y)
    return y


@functools.partial(
    pl.kernel,
    mesh=_mesh,
    out_type=jax.ShapeDtypeStruct((NP, FWH), jnp.float32),
    scratch_types=[
        pltpu.VMEM((NCH, CH), jnp.int32),
        pltpu.VMEM((NCH, CH), jnp.int32),
        pltpu.VMEM((NBUF, CH, FWH), jnp.float32),
        pltpu.VMEM((CH, FWH), jnp.float32),
        pltpu.VMEM((NPT, FWH), jnp.float32),
        pltpu.VMEM((NPT, FWH), jnp.float32),
        pltpu.VMEM((NPT, FWH), jnp.float32),
        pltpu.VMEM((NPT, FWH), jnp.float32),
        pltpu.VMEM((NPT, FWH), jnp.float32),
        pltpu.VMEM((NPT, FWH), jnp.float32),
        pltpu.VMEM((16,), jnp.float32),
        pltpu.VMEM_SHARED((NP, FWH), jnp.float32),
        pltpu.VMEM_SHARED((NP, FWH), jnp.float32),
        pltpu.SemaphoreType.DMA((NBUF,)),
        pltpu.SemaphoreType.DMA((NBUF,)),
    ],
    compiler_params=_sc_params,
)
def _prop_kernel(src_hbm, dst_hbm, y0_hbm, ones8, zero8, b2_hbm, out_hbm,
                 src_idx, dst_idx, rows, ones_t, u_t, c1_t, g_t, agg_t, sq_t,
                 zero_t, b2_t, acc, u_sp, gsem, ssem):
    w = lax.axis_index("s")
    sl = pl.ds(w * NPT, NPT)
    lanes = lax.iota(jnp.int32, 16)

    pltpu.sync_copy(src_hbm.at[w], src_idx)
    pltpu.sync_copy(dst_hbm.at[w], dst_idx)
    pltpu.sync_copy(y0_hbm.at[sl], u_t)
    pltpu.sync_copy(ones8, ones_t)
    pltpu.sync_copy(zero8, zero_t)
    pltpu.sync_copy(b2_hbm, b2_t)
    pltpu.sync_copy(zero_t, acc.at[sl])
    plsc.subcore_barrier()

    def dchunk(c, _):
        pltpu.sync_copy(ones_t, acc.at[dst_idx.at[c]], add=True)
        return 0
    lax.fori_loop(0, NCH, dchunk, 0)
    plsc.subcore_barrier()

    pltpu.sync_copy(acc.at[sl], agg_t)
    pltpu.sync_copy(zero_t, acc.at[sl])

    def pre(i, _):
        f = i * 16 + lanes
        r = lax.shift_right_logical(f, 3)
        col = lax.bitwise_and(f, 7)
        d = plsc.load_gather(agg_t, [r, col]) + 1.0
        rs = _rsqrt16(d)
        u0 = plsc.load_gather(u_t, [r, col]) * rs
        plsc.store_scatter(u_t, [r, col], u0)
        plsc.store_scatter(c1_t, [r, col], (1.0 - ALPHA) / d)
        plsc.store_scatter(g_t, [r, col], ALPHA * u0)
        plsc.store_scatter(sq_t, [r, col], d * rs)
        return 0
    lax.fori_loop(0, NV, pre, 0)

    pltpu.sync_copy(u_t, u_sp.at[sl])
    plsc.subcore_barrier()

    def step(k, _):
        for b in range(LAG):
            pltpu.async_copy(u_sp.at[src_idx.at[b]], rows.at[b], gsem.at[b])

        def ring(j, _c):
            for b in range(NBUF):
                c = j * NBUF + b
                bn = (b + LAG) % NBUF
                pltpu.make_async_copy(
                    u_sp.at[src_idx.at[c]], rows.at[b], gsem.at[b]).wait()
                pltpu.async_copy(rows.at[b], acc.at[dst_idx.at[c]],
                                 ssem.at[b], add=True)

                @pl.when((c >= NBUF - LAG) & (c + LAG < NCH))
                def _():
                    pltpu.make_async_copy(
                        rows.at[bn], acc.at[dst_idx.at[0]], ssem.at[bn]).wait()

                @pl.when(c + LAG < NCH)
                def _():
                    pltpu.async_copy(u_sp.at[src_idx.at[c + LAG]],
                                     rows.at[bn], gsem.at[bn])
            return 0
        lax.fori_loop(0, NCH // NBUF, ring, 0)
        for b in range(NBUF):
            pltpu.make_async_copy(
                rows.at[b], acc.at[dst_idx.at[0]], ssem.at[b]).wait()
        plsc.subcore_barrier()

        pltpu.sync_copy(acc.at[sl], agg_t)
        pltpu.sync_copy(zero_t, acc.at[sl])

        def upd(i, _u):
            f = i * 16 + lanes
            r = lax.shift_right_logical(f, 3)
            col = lax.bitwise_and(f, 7)
            a = plsc.load_gather(agg_t, [r, col])
            u = plsc.load_gather(u_t, [r, col])
            c1v = plsc.load_gather(c1_t, [r, col])
            gv = plsc.load_gather(g_t, [r, col])
            plsc.store_scatter(u_t, [r, col], c1v * (a + u) + gv)
            return 0
        lax.fori_loop(0, NV, upd, 0)

        pltpu.sync_copy(u_t, u_sp.at[sl])
        plsc.subcore_barrier()
        return 0

    lax.fori_loop(0, K, step, 0)

    b2v = b2_t[...]

    def fin(i, _):
        f = i * 16 + lanes
        r = lax.shift_right_logical(f, 3)
        col = lax.bitwise_and(f, 7)
        u = plsc.load_gather(u_t, [r, col])
        sq = plsc.load_gather(sq_t, [r, col])
        plsc.store_scatter(g_t, [r, col], sq * u + b2v)
        return 0
    lax.fori_loop(0, NV, fin, 0)
    pltpu.sync_copy(g_t, out_hbm.at[sl])


def kernel(x, edge_index, W1, b1, W2, b2):
    src = edge_index[0]
    dst = edge_index[1]
    pad = jnp.full((EPAD - E,), PADNODE, dtype=jnp.int32)
    src3 = jnp.concatenate([src, pad]).reshape(NT, NCH, CH)
    dst3 = jnp.concatenate([dst, pad]).reshape(NT, NCH, CH)
    x_p = jnp.pad(x, ((0, NP - N), (0, 0)))
    b1r = b1.reshape(1, H)
    W2p = jnp.pad(W2, ((0, 0), (0, FW - C)))
    b2v = jnp.tile(jnp.pad(b2, (0, FWH - C)), 2)
    zero8 = jnp.zeros((NPT, FWH), jnp.float32)
    ones8 = jnp.ones((CH, FWH), jnp.float32)

    y0 = _encoder(x_p, W1, b1r, W2p)
    out8 = _prop_kernel(src3, dst3, y0[:, :FWH], ones8, zero8, b2v)
    return out8[:N, :C]
